# Initial kernel scaffold; baseline (speedup 1.0000x reference)
import jax
import jax.numpy as jnp
from jax.experimental import pallas as pl


def kernel(qs, ds, q_table, d_table):
    raise NotImplementedError("write your pallas kernel here")


# speedup vs baseline: 2.1612x; 2.1612x over previous
import functools

import jax
import jax.numpy as jnp
from jax import lax
from jax.experimental import pallas as pl
from jax.experimental.pallas import tpu as pltpu
from jax.experimental.pallas import tpu_sc as plsc

B = 4096
L = 200
D = 128
NLANE = 16
NVREG = D // NLANE
LHALF = L // 2
LPAD = 104
NC, NS = 2, 16
NW = NC * NS
EPB = B // NW


def _accumulate(buf, acc):
    def body(r, acc):
        return tuple(acc[c] + buf[r, pl.ds(c * NLANE, NLANE)]
                     for c in range(NVREG))

    return lax.fori_loop(0, LHALF, body, acc)


def _pool_one_table(tbl_hbm, idx3_hbm, out_hbm, idx_v, buf0, buf1, stage,
                    sem0, sem1, base):
    pltpu.sync_copy(idx3_hbm.at[pl.ds(base, EPB)], idx_v)
    pltpu.async_copy(tbl_hbm.at[idx_v.at[0, 0]], buf0, sem0)
    pltpu.async_copy(tbl_hbm.at[idx_v.at[0, 1]], buf1, sem1)

    def body(g, carry):
        acc = tuple(jnp.zeros((NLANE,), jnp.float32) for _ in range(NVREG))
        pltpu.make_async_copy(tbl_hbm.at[idx_v.at[0, 0]], buf0, sem0).wait()
        acc = _accumulate(buf0, acc)

        @pl.when(g + 1 < EPB)
        def _():
            pltpu.async_copy(tbl_hbm.at[idx_v.at[g + 1, 0]], buf0, sem0)

        pltpu.make_async_copy(tbl_hbm.at[idx_v.at[0, 1]], buf1, sem1).wait()
        acc = _accumulate(buf1, acc)

        @pl.when(g + 1 < EPB)
        def _():
            pltpu.async_copy(tbl_hbm.at[idx_v.at[g + 1, 1]], buf1, sem1)

        for c in range(NVREG):
            stage[g, pl.ds(c * NLANE, NLANE)] = acc[c]
        return carry

    lax.fori_loop(0, EPB, body, 0)
    pltpu.sync_copy(stage, out_hbm.at[pl.ds(base, EPB)])


def _pool_kernel(qs_hbm, ds_hbm, qt_hbm, dt_hbm, qo_hbm, do_hbm,
                 idx_v, buf0, buf1, stage, sem0, sem1):
    wid = lax.axis_index("s") * NC + lax.axis_index("c")
    base = wid * EPB
    _pool_one_table(qt_hbm, qs_hbm, qo_hbm, idx_v, buf0, buf1, stage,
                    sem0, sem1, base)
    _pool_one_table(dt_hbm, ds_hbm, do_hbm, idx_v, buf0, buf1, stage,
                    sem0, sem1, base)


@functools.cache
def _pool():
    return pl.kernel(
        _pool_kernel,
        out_type=(
            jax.ShapeDtypeStruct((B, D), jnp.float32),
            jax.ShapeDtypeStruct((B, D), jnp.float32),
        ),
        mesh=plsc.VectorSubcoreMesh(core_axis_name="c", subcore_axis_name="s"),
        scratch_types=[
            pltpu.VMEM((EPB, 2, LPAD), jnp.int32),
            pltpu.VMEM((LPAD, D), jnp.float32),
            pltpu.VMEM((LPAD, D), jnp.float32),
            pltpu.VMEM((EPB, D), jnp.float32),
            pltpu.SemaphoreType.DMA,
            pltpu.SemaphoreType.DMA,
        ],
    )


def _tail_kernel(qsum_ref, dsum_ref, out_ref):
    scale = jnp.float32(1.0 / L)
    q = jnp.tanh(qsum_ref[...] * scale)
    d = jnp.tanh(dsum_ref[...] * scale)
    eps = jnp.float32(1e-12)
    nq = jnp.maximum(jnp.sqrt(jnp.sum(q * q, axis=1, keepdims=True)), eps)
    nd = jnp.maximum(jnp.sqrt(jnp.sum(d * d, axis=1, keepdims=True)), eps)
    num = jnp.sum(q * d, axis=1, keepdims=True)
    out_ref[...] = num / (nq * nd)


def kernel(qs, ds, q_table, d_table):
    pad = ((0, 0), (0, 0), (0, LPAD - LHALF))
    qs3 = jnp.pad(qs.reshape(B, 2, LHALF), pad)
    ds3 = jnp.pad(ds.reshape(B, 2, LHALF), pad)
    q_sum, d_sum = _pool()(qs3, ds3, q_table, d_table)
    sims = pl.pallas_call(
        _tail_kernel,
        out_shape=jax.ShapeDtypeStruct((B, 1), jnp.float32),
    )(q_sum, d_sum)
    return sims.reshape(B)


# speedup vs baseline: 2.1642x; 1.0014x over previous
import functools

import jax
import jax.numpy as jnp
from jax import lax
from jax.experimental import pallas as pl
from jax.experimental.pallas import tpu as pltpu
from jax.experimental.pallas import tpu_sc as plsc

B = 4096
L = 200
D = 128
NLANE = 16
NVREG = D // NLANE
LHALF = L // 2
LPAD = 104
NC, NS = 2, 16
NW = NC * NS
EPB = B // NW


NBUF = 4
UNROLL = 4


def _accumulate(buf, acc):
    def body(r, acc):
        for j in range(UNROLL):
            acc = tuple(acc[c] + buf[r * UNROLL + j, pl.ds(c * NLANE, NLANE)]
                        for c in range(NVREG))
        return acc

    return lax.fori_loop(0, LHALF // UNROLL, body, acc)


def _pool_one_table(tbl_hbm, idx3_hbm, out_hbm, idx_v, bufs, stage,
                    sems, base):
    pltpu.sync_copy(idx3_hbm.at[pl.ds(base, EPB)], idx_v)
    for b in range(NBUF):
        pltpu.async_copy(tbl_hbm.at[idx_v.at[b // 2, b % 2]], bufs[b], sems[b])

    ngroups = 2 * EPB // NBUF

    def body(g, carry):
        for e in range(NBUF // 2):
            acc = tuple(jnp.zeros((NLANE,), jnp.float32) for _ in range(NVREG))
            for h in range(2):
                b = 2 * e + h
                pltpu.make_async_copy(
                    tbl_hbm.at[idx_v.at[0, 0]], bufs[b], sems[b]).wait()
                acc = _accumulate(bufs[b], acc)

                @pl.when(g + 1 < ngroups)
                def _():
                    nxt = (NBUF // 2) * (g + 1) + e
                    pltpu.async_copy(
                        tbl_hbm.at[idx_v.at[nxt, h]], bufs[b], sems[b])

            elem = (NBUF // 2) * g + e
            for c in range(NVREG):
                stage[elem, pl.ds(c * NLANE, NLANE)] = acc[c]
        return carry

    lax.fori_loop(0, ngroups, body, 0)
    pltpu.sync_copy(stage, out_hbm.at[pl.ds(base, EPB)])


def _pool_kernel(qs_hbm, ds_hbm, qt_hbm, dt_hbm, qo_hbm, do_hbm,
                 idx_v, buf0, buf1, buf2, buf3, stage,
                 sem0, sem1, sem2, sem3):
    wid = lax.axis_index("s") * NC + lax.axis_index("c")
    base = wid * EPB
    bufs = (buf0, buf1, buf2, buf3)
    sems = (sem0, sem1, sem2, sem3)
    _pool_one_table(qt_hbm, qs_hbm, qo_hbm, idx_v, bufs, stage, sems, base)
    _pool_one_table(dt_hbm, ds_hbm, do_hbm, idx_v, bufs, stage, sems, base)


@functools.cache
def _pool():
    return pl.kernel(
        _pool_kernel,
        out_type=(
            jax.ShapeDtypeStruct((B, D), jnp.float32),
            jax.ShapeDtypeStruct((B, D), jnp.float32),
        ),
        mesh=plsc.VectorSubcoreMesh(core_axis_name="c", subcore_axis_name="s"),
        scratch_types=(
            [pltpu.VMEM((EPB, 2, LPAD), jnp.int32)]
            + [pltpu.VMEM((LPAD, D), jnp.float32) for _ in range(NBUF)]
            + [pltpu.VMEM((EPB, D), jnp.float32)]
            + [pltpu.SemaphoreType.DMA for _ in range(NBUF)]
        ),
    )


def _tail_kernel(qsum_ref, dsum_ref, out_ref):
    scale = jnp.float32(1.0 / L)
    q = jnp.tanh(qsum_ref[...] * scale)
    d = jnp.tanh(dsum_ref[...] * scale)
    eps = jnp.float32(1e-12)
    nq = jnp.maximum(jnp.sqrt(jnp.sum(q * q, axis=1, keepdims=True)), eps)
    nd = jnp.maximum(jnp.sqrt(jnp.sum(d * d, axis=1, keepdims=True)), eps)
    num = jnp.sum(q * d, axis=1, keepdims=True)
    out_ref[...] = num / (nq * nd)


def kernel(qs, ds, q_table, d_table):
    pad = ((0, 0), (0, 0), (0, LPAD - LHALF))
    qs3 = jnp.pad(qs.reshape(B, 2, LHALF), pad)
    ds3 = jnp.pad(ds.reshape(B, 2, LHALF), pad)
    q_sum, d_sum = _pool()(qs3, ds3, q_table, d_table)
    sims = pl.pallas_call(
        _tail_kernel,
        out_shape=jax.ShapeDtypeStruct((B, 1), jnp.float32),
    )(q_sum, d_sum)
    return sims.reshape(B)


# speedup vs baseline: 17.7173x; 8.1865x over previous
import functools

import jax
import jax.numpy as jnp
from jax import lax
from jax.experimental import pallas as pl
from jax.experimental.pallas import tpu as pltpu
from jax.experimental.pallas import tpu_sc as plsc

B = 4096
L = 200
D = 128
NLANE = 16
NVREG = D // NLANE
LHALF = L // 2
LPAD = 104
NC, NS = 2, 16
NW = NC * NS
EPB = B // NW


NBUF = 4
UNROLL = 4


def _accumulate(buf, acc):
    def body(r, acc):
        for j in range(UNROLL):
            acc = tuple(acc[c] + buf[r * UNROLL + j, pl.ds(c * NLANE, NLANE)]
                        for c in range(NVREG))
        return acc

    return lax.fori_loop(0, LHALF // UNROLL, body, acc)


def _pool_one_table(tbl_hbm, idx3_hbm, out_hbm, idx_v, bufs, stage,
                    sems, base):
    pltpu.sync_copy(idx3_hbm.at[pl.ds(base, EPB)], idx_v)
    for b in range(NBUF):
        pltpu.async_copy(tbl_hbm.at[idx_v.at[b // 2, b % 2, pl.ds(0, LHALF)]],
                         bufs[b].at[pl.ds(0, LHALF)], sems[b])

    ngroups = 2 * EPB // NBUF

    def body(g, carry):
        for e in range(NBUF // 2):
            acc = tuple(jnp.zeros((NLANE,), jnp.float32) for _ in range(NVREG))
            for h in range(2):
                b = 2 * e + h
                pltpu.make_async_copy(
                    tbl_hbm.at[idx_v.at[0, 0, pl.ds(0, LHALF)]],
                    bufs[b].at[pl.ds(0, LHALF)], sems[b]).wait()
                acc = _accumulate(bufs[b], acc)

                @pl.when(g + 1 < ngroups)
                def _():
                    nxt = (NBUF // 2) * (g + 1) + e
                    pltpu.async_copy(
                        tbl_hbm.at[idx_v.at[nxt, h, pl.ds(0, LHALF)]],
                        bufs[b].at[pl.ds(0, LHALF)], sems[b])

            elem = (NBUF // 2) * g + e
            for c in range(NVREG):
                stage[elem, pl.ds(c * NLANE, NLANE)] = acc[c]
        return carry

    lax.fori_loop(0, ngroups, body, 0)
    pltpu.sync_copy(stage, out_hbm.at[pl.ds(base, EPB)])


def _pool_kernel(qs_hbm, ds_hbm, qt_hbm, dt_hbm, qo_hbm, do_hbm,
                 idx_v, buf0, buf1, buf2, buf3, stage,
                 sem0, sem1, sem2, sem3):
    wid = lax.axis_index("s") * NC + lax.axis_index("c")
    base = wid * EPB
    bufs = (buf0, buf1, buf2, buf3)
    sems = (sem0, sem1, sem2, sem3)
    _pool_one_table(qt_hbm, qs_hbm, qo_hbm, idx_v, bufs, stage, sems, base)
    _pool_one_table(dt_hbm, ds_hbm, do_hbm, idx_v, bufs, stage, sems, base)


@functools.cache
def _pool():
    return pl.kernel(
        _pool_kernel,
        out_type=(
            jax.ShapeDtypeStruct((B, D), jnp.float32),
            jax.ShapeDtypeStruct((B, D), jnp.float32),
        ),
        mesh=plsc.VectorSubcoreMesh(core_axis_name="c", subcore_axis_name="s"),
        scratch_types=(
            [pltpu.VMEM((EPB, 2, LPAD), jnp.int32)]
            + [pltpu.VMEM((LPAD, D), jnp.float32) for _ in range(NBUF)]
            + [pltpu.VMEM((EPB, D), jnp.float32)]
            + [pltpu.SemaphoreType.DMA for _ in range(NBUF)]
        ),
    )


def _tail_kernel(qsum_ref, dsum_ref, out_ref):
    scale = jnp.float32(1.0 / L)
    q = jnp.tanh(qsum_ref[...] * scale)
    d = jnp.tanh(dsum_ref[...] * scale)
    eps = jnp.float32(1e-12)
    nq = jnp.maximum(jnp.sqrt(jnp.sum(q * q, axis=1, keepdims=True)), eps)
    nd = jnp.maximum(jnp.sqrt(jnp.sum(d * d, axis=1, keepdims=True)), eps)
    num = jnp.sum(q * d, axis=1, keepdims=True)
    out_ref[...] = num / (nq * nd)


def kernel(qs, ds, q_table, d_table):
    padv = jnp.arange(B * 2 * (LPAD - LHALF), dtype=jnp.int32)
    padv = padv.reshape(B, 2, LPAD - LHALF) % jnp.int32(q_table.shape[0])
    qs3 = jnp.concatenate([qs.reshape(B, 2, LHALF), padv], axis=2)
    ds3 = jnp.concatenate([ds.reshape(B, 2, LHALF), padv], axis=2)
    q_sum, d_sum = _pool()(qs3, ds3, q_table, d_table)
    sims = pl.pallas_call(
        _tail_kernel,
        out_shape=jax.ShapeDtypeStruct((B, 1), jnp.float32),
    )(q_sum, d_sum)
    return sims.reshape(B)
